# baseline (speedup 1.0000x reference)
import functools

import jax
import jax.numpy as jnp
from jax import lax
from jax.experimental import pallas as pl
from jax.experimental.pallas import tpu as pltpu
from jax.experimental.pallas import tpu_sc as plsc

N = 10000
E = 320000
D = 128
B = 64
NP = 10240
NT = 16
CH = 128
NB = 4
AGG_CHUNKS = 160
EP = NT * AGG_CHUNKS * CH
DEG_CHUNKS = 157
DP = NT * DEG_CHUNKS * CH

_mesh = plsc.VectorSubcoreMesh(core_axis_name="c", subcore_axis_name="s")


def _deg_body(didx_hbm, out_hbm, idx_v, ones_v, zeros_v, acc_sh, sem):
    c = lax.axis_index("c")
    t = lax.axis_index("s")

    def _fill(i, _):
        zeros_v[pl.ds(i * 16, 16)] = jnp.zeros((16,), jnp.float32)
        return 0

    lax.fori_loop(0, 80, _fill, 0)
    for j in range(8):
        ones_v[pl.ds(j * 16, 16)] = jnp.full((16,), 1.0, jnp.float32)
    pltpu.sync_copy(zeros_v, acc_sh.at[pl.ds(t * 1280, 1280)])
    pltpu.sync_copy(didx_hbm.at[c, t], idx_v)
    plsc.subcore_barrier()

    def _scat(j, _):
        pltpu.async_copy(ones_v, acc_sh.at[idx_v.at[j]], sem, add=True)
        return 0

    lax.fori_loop(0, 2 * DEG_CHUNKS, _scat, 0)

    def _drain(j, _):
        pltpu.make_async_copy(ones_v, acc_sh.at[idx_v.at[0]], sem).wait()
        return 0

    lax.fori_loop(0, 2 * DEG_CHUNKS, _drain, 0)
    plsc.subcore_barrier()

    @pl.when(t == 0)
    def _():
        pltpu.sync_copy(acc_sh, out_hbm.at[c])


_deg_kernel = pl.kernel(
    _deg_body,
    out_type=jax.ShapeDtypeStruct((2, 2 * NP), jnp.float32),
    mesh=_mesh,
    scratch_types=[
        pltpu.VMEM((2 * DEG_CHUNKS, CH), jnp.int32),
        pltpu.VMEM((CH,), jnp.float32),
        pltpu.VMEM((1280,), jnp.float32),
        pltpu.VMEM_SHARED((2 * NP,), jnp.float32),
        pltpu.SemaphoreType.DMA,
    ],
)


HD = D // 2


def _agg_body(xn_hbm, src_hbm, dst_hbm, out_hbm, src_v, dst_v, rows_v,
              zbuf_v, acc_sh, gsem, ssem):
    c = lax.axis_index("c")
    t = lax.axis_index("s")

    def _zrow(i, _):
        for j in range(HD // 16):
            zbuf_v[i, pl.ds(j * 16, 16)] = jnp.zeros((16,), jnp.float32)
        return 0

    lax.fori_loop(0, CH, _zrow, 0)
    pltpu.sync_copy(src_hbm.at[c, t], src_v)
    pltpu.sync_copy(dst_hbm.at[c, t], dst_v)

    def _zero_acc():
        for k in range(5):
            pltpu.sync_copy(zbuf_v,
                            acc_sh.at[pl.ds(t * 640 + k * CH, CH)])

    def _edge_pass():
        for b in range(2):
            pltpu.async_copy(xn_hbm.at[src_v.at[b]], rows_v.at[b], gsem.at[b])

        def _step(k, _):
            for b in range(NB):
                j = NB * k + b
                pltpu.make_async_copy(
                    xn_hbm.at[src_v.at[j]], rows_v.at[b], gsem.at[b]).wait()
                pltpu.async_copy(rows_v.at[b], acc_sh.at[dst_v.at[j]],
                                 ssem.at[b], add=True)
                b2 = (b + 2) % NB
                if b < 2:

                    @pl.when(k > 0)
                    def _():
                        pltpu.make_async_copy(
                            rows_v.at[b2], acc_sh.at[dst_v.at[j]],
                            ssem.at[b2]).wait()

                    pltpu.async_copy(
                        xn_hbm.at[src_v.at[j + 2]], rows_v.at[b2],
                        gsem.at[b2])
                else:
                    pltpu.make_async_copy(
                        rows_v.at[b2], acc_sh.at[dst_v.at[j]],
                        ssem.at[b2]).wait()

                    @pl.when(k < AGG_CHUNKS // NB - 1)
                    def _():
                        pltpu.async_copy(
                            xn_hbm.at[src_v.at[j + 2]], rows_v.at[b2],
                            gsem.at[b2])
            return 0

        lax.fori_loop(0, AGG_CHUNKS // NB, _step, 0)
        for b in (2, 3):
            pltpu.make_async_copy(rows_v.at[b], acc_sh.at[dst_v.at[0]],
                                  ssem.at[b]).wait()

    def _writeback(p):
        for k in range(5):
            pltpu.sync_copy(acc_sh.at[pl.ds(t * 640 + k * CH, CH)],
                            out_hbm.at[p, c, pl.ds(t * 640 + k * CH, CH)])

    def _shift_src():
        def _sh(i, _):
            for j in range(CH // 16):
                sl = pl.ds(j * 16, 16)
                src_v[i, sl] = src_v[i, sl] + jnp.full((16,), 2 * NP,
                                                       jnp.int32)
            return 0

        lax.fori_loop(0, AGG_CHUNKS, _sh, 0)

    _zero_acc()
    plsc.subcore_barrier()
    _edge_pass()
    plsc.subcore_barrier()
    _writeback(0)
    _shift_src()
    _zero_acc()
    plsc.subcore_barrier()
    _edge_pass()
    plsc.subcore_barrier()
    _writeback(1)


_agg_kernel = pl.kernel(
    _agg_body,
    out_type=jax.ShapeDtypeStruct((2, 2, NP, HD), jnp.float32),
    mesh=_mesh,
    scratch_types=[
        pltpu.VMEM((AGG_CHUNKS, CH), jnp.int32),
        pltpu.VMEM((AGG_CHUNKS, CH), jnp.int32),
        pltpu.VMEM((NB, CH, HD), jnp.float32),
        pltpu.VMEM((CH, HD), jnp.float32),
        pltpu.VMEM_SHARED((NP, HD), jnp.float32),
        pltpu.SemaphoreType.DMA((NB,)),
        pltpu.SemaphoreType.DMA((NB,)),
    ],
    compiler_params=pltpu.CompilerParams(use_tc_tiling_on_sc=False),
)


_RB = 1024


def _prep_body(x_ref, dego_ref, degi_ref, xn_ref, ri_ref, ro_ref):
    ro = lax.rsqrt(jnp.maximum(dego_ref[0, 0, 0], 1.0))
    ri_ref[0, 0, 0] = lax.rsqrt(jnp.maximum(degi_ref[0, 0, 0], 1.0))
    ro_ref[0, 0, 0] = ro
    xn = x_ref[0] * ro[:, None]
    xn_ref[0, 0] = xn[:, :HD]
    xn_ref[1, 0] = xn[:, HD:]


_prep_call = pl.pallas_call(
    _prep_body,
    grid=(2, NP // _RB),
    in_specs=[
        pl.BlockSpec((1, _RB, D), lambda g, i: (g, i, 0)),
        pl.BlockSpec((1, 1, 1, _RB), lambda g, i: (g, i, 0, 0)),
        pl.BlockSpec((1, 1, 1, _RB), lambda g, i: (g, i, 0, 0)),
    ],
    out_specs=[
        pl.BlockSpec((2, 1, _RB, HD), lambda g, i: (0, g, i, 0)),
        pl.BlockSpec((1, 1, 1, _RB), lambda g, i: (g, i, 0, 0)),
        pl.BlockSpec((1, 1, 1, _RB), lambda g, i: (g, i, 0, 0)),
    ],
    out_shape=[
        jax.ShapeDtypeStruct((2, 2, NP, HD), jnp.float32),
        jax.ShapeDtypeStruct((2, NP // _RB, 1, _RB), jnp.float32),
        jax.ShapeDtypeStruct((2, NP // _RB, 1, _RB), jnp.float32),
    ],
)


def _mid_body(lo_ref, hi_ref, ri_ref, sc_ref, w_ref, b_ref, out_ref):
    ri = ri_ref[...]
    w = w_ref[...]
    h = (jnp.dot(lo_ref[...] * ri, w[:HD],
                 preferred_element_type=jnp.float32)
         + jnp.dot(hi_ref[...] * ri, w[HD:],
                   preferred_element_type=jnp.float32))
    h = jnp.maximum(h + b_ref[...], 0.0) * sc_ref[...]
    out_ref[0] = h[:, :HD]
    out_ref[1] = h[:, HD:]


_mid_call = pl.pallas_call(
    _mid_body,
    grid=(2 * NP // _RB,),
    in_specs=[
        pl.BlockSpec((_RB, HD), lambda i: (i, 0)),
        pl.BlockSpec((_RB, HD), lambda i: (i, 0)),
        pl.BlockSpec((_RB, 1), lambda i: (i, 0)),
        pl.BlockSpec((_RB, 1), lambda i: (i, 0)),
        pl.BlockSpec((D, D), lambda i: (0, 0)),
        pl.BlockSpec((1, D), lambda i: (0, 0)),
    ],
    out_specs=pl.BlockSpec((2, _RB, HD), lambda i: (0, i, 0)),
    out_shape=jax.ShapeDtypeStruct((2, 2 * NP, HD), jnp.float32),
)


def _fin_body(h_ref, g1_ref, g2_ref, desc_ref,
              c1w_ref, c1b_ref, c2w_ref, c2b_ref, c3w_ref, c3b_ref,
              c4w_ref, c4b_ref, out_ref):
    iota = lax.broadcasted_iota(jnp.int32, (1, B), 1)

    def pool(g_ref, rows):
        m = (g_ref[...] == iota).astype(jnp.float32)
        s = lax.dot_general(m, rows, (((0,), (0,)), ((), ())),
                            preferred_element_type=jnp.float32)
        cnt = jnp.sum(m, axis=0)[:, None]
        return s / jnp.maximum(cnt, 1.0)

    hg1 = pool(g1_ref, jnp.concatenate(
        [h_ref[0:N], h_ref[2 * NP:2 * NP + N]], axis=1))
    hg2 = pool(g2_ref, jnp.concatenate(
        [h_ref[NP:NP + N], h_ref[3 * NP:3 * NP + N]], axis=1))

    c1w = c1w_ref[...]
    z = (jnp.dot(hg1, c1w[0:D], preferred_element_type=jnp.float32)
         + jnp.dot(hg2, c1w[D:2 * D], preferred_element_type=jnp.float32)
         + jnp.dot(desc_ref[...], c1w[2 * D:], preferred_element_type=jnp.float32)
         + c1b_ref[...])
    z = jnp.maximum(z, 0.0)
    z = jnp.maximum(jnp.dot(z, c2w_ref[...],
                            preferred_element_type=jnp.float32) + c2b_ref[...], 0.0)
    z = jnp.maximum(jnp.dot(z, c3w_ref[...],
                            preferred_element_type=jnp.float32) + c3b_ref[...], 0.0)
    out_ref[...] = jnp.dot(z, c4w_ref[...],
                           preferred_element_type=jnp.float32) + c4b_ref[...]


_fin_call = pl.pallas_call(
    _fin_body,
    out_shape=jax.ShapeDtypeStruct((B, 1), jnp.float32),
)


def _prep_deg_idx(ei):
    src, dst = ei[0], ei[1]
    padn = DP - E
    spread = jnp.arange(padn, dtype=jnp.int32) % (NP - N)
    s = jnp.concatenate([src, N + spread]).reshape(NT, DEG_CHUNKS, CH)
    d = jnp.concatenate([dst + NP, NP + N + spread]).reshape(NT, DEG_CHUNKS, CH)
    return jnp.concatenate([s, d], axis=1)


def _prep_agg_idx(ei, g):
    src, dst = ei[0], ei[1]
    padn = EP - E
    pad_src = jnp.arange(padn, dtype=jnp.int32) % N
    pad_dst = N + (jnp.arange(padn, dtype=jnp.int32) % (NP - N))
    s = (jnp.concatenate([src, pad_src]) + g * NP).reshape(NT, AGG_CHUNKS, CH)
    d = jnp.concatenate([dst, pad_dst]).reshape(NT, AGG_CHUNKS, CH)
    return s, d


def kernel(x1, x2, edge_index1, edge_index2, graph_ids1, graph_ids2,
           descriptors, W1, b1, W2, b2, C1W, C1b, C2W, C2b, C3W, C3b,
           C4W, C4b):
    didx = jnp.stack([_prep_deg_idx(edge_index1), _prep_deg_idx(edge_index2)])
    deg = _deg_kernel(didx)

    xpad = jnp.pad(jnp.stack([x1, x2]), ((0, 0), (0, NP - N), (0, 0)))
    dego4 = deg[:, :NP].reshape(2, NP // _RB, 1, _RB)
    degi4 = deg[:, NP:].reshape(2, NP // _RB, 1, _RB)
    xn4, ri4, ro4 = _prep_call(xpad, dego4, degi4)
    xn = xn4.reshape(4 * NP, HD)
    ri = ri4.reshape(2 * NP, 1)
    ro = ro4.reshape(2 * NP, 1)

    s1, d1 = _prep_agg_idx(edge_index1, 0)
    s2, d2 = _prep_agg_idx(edge_index2, 1)
    srcs = jnp.stack([s1, s2])
    dsts = jnp.stack([d1, d2])

    wl = jnp.stack([W1, W2])
    bl = jnp.stack([b1.reshape(1, D), b2.reshape(1, D)])
    sc = jnp.stack([ro, jnp.ones_like(ro)])

    def layer(h, per):
        w, b_, s_ = per
        agg = _agg_kernel(h, srcs, dsts)
        lo = agg[0].reshape(2 * NP, HD)
        hi = agg[1].reshape(2 * NP, HD)
        return _mid_call(lo, hi, ri, s_, w, b_).reshape(4 * NP, HD), None

    h, _ = lax.scan(layer, xn, (wl, bl, sc))

    return _fin_call(h, graph_ids1.reshape(N, 1), graph_ids2.reshape(N, 1),
                     descriptors, C1W, C1b.reshape(1, 2 * D + 16),
                     C2W, C2b.reshape(1, D), C3W, C3b.reshape(1, D),
                     C4W, C4b.reshape(1, 1))


# speedup vs baseline: 1.6226x; 1.6226x over previous
import functools

import jax
import jax.numpy as jnp
from jax import lax
from jax.experimental import pallas as pl
from jax.experimental.pallas import tpu as pltpu
from jax.experimental.pallas import tpu_sc as plsc

N = 10000
E = 320000
D = 128
B = 64
NP = 10240
NT = 16
CH = 128
NB = 4
AGG_CHUNKS = 160
EP = NT * AGG_CHUNKS * CH
DEG_CHUNKS = 157
DP = NT * DEG_CHUNKS * CH

_mesh = plsc.VectorSubcoreMesh(core_axis_name="c", subcore_axis_name="s")


def _deg_body(didx_hbm, out_hbm, idx_v, ones_v, zeros_v, acc_sh, sem):
    c = lax.axis_index("c")
    t = lax.axis_index("s")

    def _fill(i, _):
        zeros_v[pl.ds(i * 16, 16)] = jnp.zeros((16,), jnp.float32)
        return 0

    lax.fori_loop(0, 80, _fill, 0)
    for j in range(8):
        ones_v[pl.ds(j * 16, 16)] = jnp.full((16,), 1.0, jnp.float32)
    pltpu.sync_copy(zeros_v, acc_sh.at[pl.ds(t * 1280, 1280)])
    pltpu.sync_copy(didx_hbm.at[c, t], idx_v)
    plsc.subcore_barrier()

    def _scat(j, _):
        pltpu.sync_copy(ones_v, acc_sh.at[idx_v.at[j]], add=True)
        return 0

    lax.fori_loop(0, 2 * DEG_CHUNKS, _scat, 0)
    plsc.subcore_barrier()

    @pl.when(t == 0)
    def _():
        pltpu.sync_copy(acc_sh, out_hbm.at[c])


_deg_kernel = pl.kernel(
    _deg_body,
    out_type=jax.ShapeDtypeStruct((2, 2 * NP), jnp.float32),
    mesh=_mesh,
    scratch_types=[
        pltpu.VMEM((2 * DEG_CHUNKS, CH), jnp.int32),
        pltpu.VMEM((CH,), jnp.float32),
        pltpu.VMEM((1280,), jnp.float32),
        pltpu.VMEM_SHARED((2 * NP,), jnp.float32),
        pltpu.SemaphoreType.DMA,
    ],
)


def _agg_body(xn_hbm, src_hbm, dst_hbm, out_hbm, src_v, dst_v, rows_v,
              zbuf_v, acc_sh, gsem):
    c = lax.axis_index("c")
    t = lax.axis_index("s")

    def _zrow(i, _):
        for j in range(D // 32):
            zbuf_v[i, pl.ds(j * 32, 32)] = jnp.zeros((32,), jnp.bfloat16)
        return 0

    lax.fori_loop(0, CH, _zrow, 0)
    pltpu.sync_copy(src_hbm.at[c, t], src_v)
    pltpu.sync_copy(dst_hbm.at[c, t], dst_v)
    for k in range(5):
        pltpu.sync_copy(zbuf_v, acc_sh.at[pl.ds(t * 640 + k * CH, CH)])
    plsc.subcore_barrier()

    for b in range(NB):
        pltpu.async_copy(xn_hbm.at[src_v.at[b]], rows_v.at[b], gsem.at[b])

    def _step(k, _):
        for b in range(NB):
            j = NB * k + b
            pltpu.make_async_copy(
                xn_hbm.at[src_v.at[j]], rows_v.at[b], gsem.at[b]).wait()
            pltpu.sync_copy(rows_v.at[b], acc_sh.at[dst_v.at[j]], add=True)

            @pl.when(k < AGG_CHUNKS // NB - 1)
            def _():
                pltpu.async_copy(
                    xn_hbm.at[src_v.at[NB * (k + 1) + b]], rows_v.at[b],
                    gsem.at[b])
        return 0

    lax.fori_loop(0, AGG_CHUNKS // NB, _step, 0)
    plsc.subcore_barrier()
    for k in range(5):
        pltpu.sync_copy(acc_sh.at[pl.ds(t * 640 + k * CH, CH)],
                        out_hbm.at[c, pl.ds(t * 640 + k * CH, CH)])


_agg_kernel = pl.kernel(
    _agg_body,
    out_type=jax.ShapeDtypeStruct((2, NP, D), jnp.bfloat16),
    mesh=_mesh,
    scratch_types=[
        pltpu.VMEM((AGG_CHUNKS, CH), jnp.int32),
        pltpu.VMEM((AGG_CHUNKS, CH), jnp.int32),
        pltpu.VMEM((NB, CH, D), jnp.bfloat16),
        pltpu.VMEM((CH, D), jnp.bfloat16),
        pltpu.VMEM_SHARED((NP, D), jnp.bfloat16),
        pltpu.SemaphoreType.DMA((NB,)),
    ],
    compiler_params=pltpu.CompilerParams(use_tc_tiling_on_sc=False),
)


_RB = 1024


def _prep_body(x_ref, dego_ref, degi_ref, xn_ref, ri_ref, ro_ref):
    ro = lax.rsqrt(jnp.maximum(dego_ref[0, 0, 0], 1.0))
    ri_ref[0, 0, 0] = lax.rsqrt(jnp.maximum(degi_ref[0, 0, 0], 1.0))
    ro_ref[0, 0, 0] = ro
    xn_ref[0] = (x_ref[0] * ro[:, None]).astype(jnp.bfloat16)


_prep_call = pl.pallas_call(
    _prep_body,
    grid=(2, NP // _RB),
    in_specs=[
        pl.BlockSpec((1, _RB, D), lambda g, i: (g, i, 0)),
        pl.BlockSpec((1, 1, 1, _RB), lambda g, i: (g, i, 0, 0)),
        pl.BlockSpec((1, 1, 1, _RB), lambda g, i: (g, i, 0, 0)),
    ],
    out_specs=[
        pl.BlockSpec((1, _RB, D), lambda g, i: (g, i, 0)),
        pl.BlockSpec((1, 1, 1, _RB), lambda g, i: (g, i, 0, 0)),
        pl.BlockSpec((1, 1, 1, _RB), lambda g, i: (g, i, 0, 0)),
    ],
    out_shape=[
        jax.ShapeDtypeStruct((2, NP, D), jnp.bfloat16),
        jax.ShapeDtypeStruct((2, NP // _RB, 1, _RB), jnp.float32),
        jax.ShapeDtypeStruct((2, NP // _RB, 1, _RB), jnp.float32),
    ],
)


def _mid_body(agg_ref, ri_ref, sc_ref, w_ref, b_ref, out_ref):
    a = agg_ref[...].astype(jnp.float32) * ri_ref[...]
    h = jnp.dot(a, w_ref[...], preferred_element_type=jnp.float32)
    h = jnp.maximum(h + b_ref[...], 0.0) * sc_ref[...]
    out_ref[...] = h.astype(jnp.bfloat16)


_mid_call = pl.pallas_call(
    _mid_body,
    grid=(2 * NP // _RB,),
    in_specs=[
        pl.BlockSpec((_RB, D), lambda i: (i, 0)),
        pl.BlockSpec((_RB, 1), lambda i: (i, 0)),
        pl.BlockSpec((_RB, 1), lambda i: (i, 0)),
        pl.BlockSpec((D, D), lambda i: (0, 0)),
        pl.BlockSpec((1, D), lambda i: (0, 0)),
    ],
    out_specs=pl.BlockSpec((_RB, D), lambda i: (i, 0)),
    out_shape=jax.ShapeDtypeStruct((2 * NP, D), jnp.bfloat16),
)


def _fin_body(h_ref, g1_ref, g2_ref, desc_ref,
              c1w_ref, c1b_ref, c2w_ref, c2b_ref, c3w_ref, c3b_ref,
              c4w_ref, c4b_ref, out_ref):
    iota = lax.broadcasted_iota(jnp.int32, (1, B), 1)

    def pool(g_ref, rows):
        m = (g_ref[...] == iota).astype(jnp.float32)
        s = lax.dot_general(m, rows, (((0,), (0,)), ((), ())),
                            preferred_element_type=jnp.float32)
        cnt = jnp.sum(m, axis=0)[:, None]
        return s / jnp.maximum(cnt, 1.0)

    hg1 = pool(g1_ref, h_ref[0:N].astype(jnp.float32))
    hg2 = pool(g2_ref, h_ref[NP:NP + N].astype(jnp.float32))

    c1w = c1w_ref[...]
    z = (jnp.dot(hg1, c1w[0:D], preferred_element_type=jnp.float32)
         + jnp.dot(hg2, c1w[D:2 * D], preferred_element_type=jnp.float32)
         + jnp.dot(desc_ref[...], c1w[2 * D:], preferred_element_type=jnp.float32)
         + c1b_ref[...])
    z = jnp.maximum(z, 0.0)
    z = jnp.maximum(jnp.dot(z, c2w_ref[...],
                            preferred_element_type=jnp.float32) + c2b_ref[...], 0.0)
    z = jnp.maximum(jnp.dot(z, c3w_ref[...],
                            preferred_element_type=jnp.float32) + c3b_ref[...], 0.0)
    out_ref[...] = jnp.dot(z, c4w_ref[...],
                           preferred_element_type=jnp.float32) + c4b_ref[...]


_fin_call = pl.pallas_call(
    _fin_body,
    out_shape=jax.ShapeDtypeStruct((B, 1), jnp.float32),
)


def _prep_deg_idx(ei):
    src, dst = ei[0], ei[1]
    padn = DP - E
    spread = jnp.arange(padn, dtype=jnp.int32) % (NP - N)
    s = jnp.concatenate([src, N + spread]).reshape(NT, DEG_CHUNKS, CH)
    d = jnp.concatenate([dst + NP, NP + N + spread]).reshape(NT, DEG_CHUNKS, CH)
    return jnp.concatenate([s, d], axis=1)


def _prep_agg_idx(ei, g):
    src, dst = ei[0], ei[1]
    padn = EP - E
    pad_src = jnp.arange(padn, dtype=jnp.int32) % N
    pad_dst = N + (jnp.arange(padn, dtype=jnp.int32) % (NP - N))
    s = (jnp.concatenate([src, pad_src]) + g * NP).reshape(NT, AGG_CHUNKS, CH)
    d = jnp.concatenate([dst, pad_dst]).reshape(NT, AGG_CHUNKS, CH)
    return s, d


def kernel(x1, x2, edge_index1, edge_index2, graph_ids1, graph_ids2,
           descriptors, W1, b1, W2, b2, C1W, C1b, C2W, C2b, C3W, C3b,
           C4W, C4b):
    didx = jnp.stack([_prep_deg_idx(edge_index1), _prep_deg_idx(edge_index2)])
    deg = _deg_kernel(didx)

    xpad = jnp.pad(jnp.stack([x1, x2]), ((0, 0), (0, NP - N), (0, 0)))
    dego4 = deg[:, :NP].reshape(2, NP // _RB, 1, _RB)
    degi4 = deg[:, NP:].reshape(2, NP // _RB, 1, _RB)
    xn3, ri4, ro4 = _prep_call(xpad, dego4, degi4)
    xn = xn3.reshape(2 * NP, D)
    ri = ri4.reshape(2 * NP, 1)
    ro = ro4.reshape(2 * NP, 1)

    s1, d1 = _prep_agg_idx(edge_index1, 0)
    s2, d2 = _prep_agg_idx(edge_index2, 1)
    srcs = jnp.stack([s1, s2])
    dsts = jnp.stack([d1, d2])

    wl = jnp.stack([W1, W2])
    bl = jnp.stack([b1.reshape(1, D), b2.reshape(1, D)])
    sc = jnp.stack([ro, jnp.ones_like(ro)])

    def layer(h, per):
        w, b_, s_ = per
        agg = _agg_kernel(h, srcs, dsts).reshape(2 * NP, D)
        return _mid_call(agg, ri, s_, w, b_), None

    h, _ = lax.scan(layer, xn, (wl, bl, sc))

    return _fin_call(h, graph_ids1.reshape(N, 1), graph_ids2.reshape(N, 1),
                     descriptors, C1W, C1b.reshape(1, 2 * D + 16),
                     C2W, C2b.reshape(1, D), C3W, C3b.reshape(1, D),
                     C4W, C4b.reshape(1, 1))


# speedup vs baseline: 1.7620x; 1.0859x over previous
import functools

import jax
import jax.numpy as jnp
from jax import lax
from jax.experimental import pallas as pl
from jax.experimental.pallas import tpu as pltpu
from jax.experimental.pallas import tpu_sc as plsc

N = 10000
E = 320000
D = 128
B = 64
NP = 10240
NT = 16
CH = 128
NB = 4
AGG_CHUNKS = 160
EP = NT * AGG_CHUNKS * CH
DEG_CHUNKS = 157
DP = NT * DEG_CHUNKS * CH

_mesh = plsc.VectorSubcoreMesh(core_axis_name="c", subcore_axis_name="s")


def _deg_body(src_hbm, dst_hbm, out_hbm, idx_v, ones_v, zeros_v,
              sdo_sh, sdi_sh, sem):
    del sem
    c = lax.axis_index("c")
    t = lax.axis_index("s")

    def _fill(i, _):
        zeros_v[pl.ds(i * 16, 16)] = jnp.zeros((16,), jnp.float32)
        return 0

    lax.fori_loop(0, 80, _fill, 0)
    for j in range(8):
        ones_v[pl.ds(j * 16, 16)] = jnp.full((16,), 1.0, jnp.float32)
    pltpu.sync_copy(zeros_v, sdo_sh.at[pl.ds(t * 1280, 1280)])
    pltpu.sync_copy(zeros_v.at[pl.ds(0, 640)], sdi_sh.at[pl.ds(t * 640, 640)])
    pltpu.sync_copy(src_hbm.at[c, t], idx_v)
    plsc.subcore_barrier()

    def _scat_o(j, _):
        pltpu.sync_copy(ones_v, sdo_sh.at[idx_v.at[j]], add=True)
        return 0

    lax.fori_loop(0, AGG_CHUNKS, _scat_o, 0)
    pltpu.sync_copy(dst_hbm.at[c, t], idx_v)

    def _scat_i(j, _):
        pltpu.sync_copy(ones_v, sdi_sh.at[idx_v.at[j]], add=True)
        return 0

    lax.fori_loop(0, AGG_CHUNKS, _scat_i, 0)
    plsc.subcore_barrier()

    @pl.when(t == 0)
    def _():
        pltpu.sync_copy(sdo_sh.at[pl.ds(c * NP, NP)],
                        out_hbm.at[c, pl.ds(0, NP)])

    @pl.when(t == 1)
    def _():
        pltpu.sync_copy(sdi_sh, out_hbm.at[c, pl.ds(NP, NP)])


_deg_kernel = pl.kernel(
    _deg_body,
    out_type=jax.ShapeDtypeStruct((2, 2 * NP), jnp.float32),
    mesh=_mesh,
    scratch_types=[
        pltpu.VMEM((AGG_CHUNKS, CH), jnp.int32),
        pltpu.VMEM((CH,), jnp.float32),
        pltpu.VMEM((1280,), jnp.float32),
        pltpu.VMEM_SHARED((2 * NP,), jnp.float32),
        pltpu.VMEM_SHARED((NP,), jnp.float32),
        pltpu.SemaphoreType.DMA,
    ],
)


def _agg_body(xn_hbm, src_hbm, dst_hbm, out_hbm, src_v, dst_v, rows_v,
              zbuf_v, acc_sh, gsem):
    c = lax.axis_index("c")
    t = lax.axis_index("s")

    def _zrow(i, _):
        for j in range(D // 32):
            zbuf_v[i, pl.ds(j * 32, 32)] = jnp.zeros((32,), jnp.bfloat16)
        return 0

    lax.fori_loop(0, CH, _zrow, 0)
    pltpu.sync_copy(src_hbm.at[c, t], src_v)
    pltpu.sync_copy(dst_hbm.at[c, t], dst_v)
    for k in range(5):
        pltpu.sync_copy(zbuf_v, acc_sh.at[pl.ds(t * 640 + k * CH, CH)])
    plsc.subcore_barrier()

    for b in range(NB):
        pltpu.async_copy(xn_hbm.at[src_v.at[b]], rows_v.at[b], gsem.at[b])

    def _step(k, _):
        for b in range(NB):
            j = NB * k + b
            pltpu.make_async_copy(
                xn_hbm.at[src_v.at[j]], rows_v.at[b], gsem.at[b]).wait()
            pltpu.sync_copy(rows_v.at[b], acc_sh.at[dst_v.at[j]], add=True)

            @pl.when(k < AGG_CHUNKS // NB - 1)
            def _():
                pltpu.async_copy(
                    xn_hbm.at[src_v.at[NB * (k + 1) + b]], rows_v.at[b],
                    gsem.at[b])
        return 0

    lax.fori_loop(0, AGG_CHUNKS // NB, _step, 0)
    plsc.subcore_barrier()
    for k in range(5):
        pltpu.sync_copy(acc_sh.at[pl.ds(t * 640 + k * CH, CH)],
                        out_hbm.at[c, pl.ds(t * 640 + k * CH, CH)])


_agg_kernel = pl.kernel(
    _agg_body,
    out_type=jax.ShapeDtypeStruct((2, NP, D), jnp.bfloat16),
    mesh=_mesh,
    scratch_types=[
        pltpu.VMEM((AGG_CHUNKS, CH), jnp.int32),
        pltpu.VMEM((AGG_CHUNKS, CH), jnp.int32),
        pltpu.VMEM((NB, CH, D), jnp.bfloat16),
        pltpu.VMEM((CH, D), jnp.bfloat16),
        pltpu.VMEM_SHARED((NP, D), jnp.bfloat16),
        pltpu.SemaphoreType.DMA((NB,)),
    ],
    compiler_params=pltpu.CompilerParams(use_tc_tiling_on_sc=False),
)


_RB = 1024


def _prep_body(x_ref, dego_ref, degi_ref, xn_ref, ri_ref, ro_ref):
    ro = lax.rsqrt(jnp.maximum(dego_ref[0, 0, 0], 1.0))
    ri_ref[0, 0, 0] = lax.rsqrt(jnp.maximum(degi_ref[0, 0, 0], 1.0))
    ro_ref[0, 0, 0] = ro
    xn_ref[0] = (x_ref[0] * ro[:, None]).astype(jnp.bfloat16)


_prep_call = pl.pallas_call(
    _prep_body,
    grid=(2, NP // _RB),
    in_specs=[
        pl.BlockSpec((1, _RB, D), lambda g, i: (g, i, 0)),
        pl.BlockSpec((1, 1, 1, _RB), lambda g, i: (g, i, 0, 0)),
        pl.BlockSpec((1, 1, 1, _RB), lambda g, i: (g, i, 0, 0)),
    ],
    out_specs=[
        pl.BlockSpec((1, _RB, D), lambda g, i: (g, i, 0)),
        pl.BlockSpec((1, 1, 1, _RB), lambda g, i: (g, i, 0, 0)),
        pl.BlockSpec((1, 1, 1, _RB), lambda g, i: (g, i, 0, 0)),
    ],
    out_shape=[
        jax.ShapeDtypeStruct((2, NP, D), jnp.bfloat16),
        jax.ShapeDtypeStruct((2, NP // _RB, 1, _RB), jnp.float32),
        jax.ShapeDtypeStruct((2, NP // _RB, 1, _RB), jnp.float32),
    ],
)


def _mid_body(agg_ref, ri_ref, sc_ref, w_ref, b_ref, out_ref):
    a = agg_ref[...].astype(jnp.float32) * ri_ref[...]
    h = jnp.dot(a, w_ref[...], preferred_element_type=jnp.float32)
    h = jnp.maximum(h + b_ref[...], 0.0) * sc_ref[...]
    out_ref[...] = h.astype(jnp.bfloat16)


_mid_call = pl.pallas_call(
    _mid_body,
    grid=(2 * NP // _RB,),
    in_specs=[
        pl.BlockSpec((_RB, D), lambda i: (i, 0)),
        pl.BlockSpec((_RB, 1), lambda i: (i, 0)),
        pl.BlockSpec((_RB, 1), lambda i: (i, 0)),
        pl.BlockSpec((D, D), lambda i: (0, 0)),
        pl.BlockSpec((1, D), lambda i: (0, 0)),
    ],
    out_specs=pl.BlockSpec((_RB, D), lambda i: (i, 0)),
    out_shape=jax.ShapeDtypeStruct((2 * NP, D), jnp.bfloat16),
)


def _fin_body(h_ref, g1_ref, g2_ref, desc_ref,
              c1w_ref, c1b_ref, c2w_ref, c2b_ref, c3w_ref, c3b_ref,
              c4w_ref, c4b_ref, out_ref):
    iota = lax.broadcasted_iota(jnp.int32, (1, B), 1)

    def pool(g_ref, rows):
        m = (g_ref[...] == iota).astype(jnp.float32)
        s = lax.dot_general(m, rows, (((0,), (0,)), ((), ())),
                            preferred_element_type=jnp.float32)
        cnt = jnp.sum(m, axis=0)[:, None]
        return s / jnp.maximum(cnt, 1.0)

    hg1 = pool(g1_ref, h_ref[0:N].astype(jnp.float32))
    hg2 = pool(g2_ref, h_ref[NP:NP + N].astype(jnp.float32))

    c1w = c1w_ref[...]
    z = (jnp.dot(hg1, c1w[0:D], preferred_element_type=jnp.float32)
         + jnp.dot(hg2, c1w[D:2 * D], preferred_element_type=jnp.float32)
         + jnp.dot(desc_ref[...], c1w[2 * D:], preferred_element_type=jnp.float32)
         + c1b_ref[...])
    z = jnp.maximum(z, 0.0)
    z = jnp.maximum(jnp.dot(z, c2w_ref[...],
                            preferred_element_type=jnp.float32) + c2b_ref[...], 0.0)
    z = jnp.maximum(jnp.dot(z, c3w_ref[...],
                            preferred_element_type=jnp.float32) + c3b_ref[...], 0.0)
    out_ref[...] = jnp.dot(z, c4w_ref[...],
                           preferred_element_type=jnp.float32) + c4b_ref[...]


_fin_call = pl.pallas_call(
    _fin_body,
    out_shape=jax.ShapeDtypeStruct((B, 1), jnp.float32),
)


def _prep_agg_idx(ei, g):
    src, dst = ei[0], ei[1]
    padn = EP - E
    spread = N + (jnp.arange(padn, dtype=jnp.int32) % (NP - N))
    s = (jnp.concatenate([src, spread]) + g * NP).reshape(NT, AGG_CHUNKS, CH)
    d = jnp.concatenate([dst, spread]).reshape(NT, AGG_CHUNKS, CH)
    return s, d


def kernel(x1, x2, edge_index1, edge_index2, graph_ids1, graph_ids2,
           descriptors, W1, b1, W2, b2, C1W, C1b, C2W, C2b, C3W, C3b,
           C4W, C4b):
    s1, d1 = _prep_agg_idx(edge_index1, 0)
    s2, d2 = _prep_agg_idx(edge_index2, 1)
    srcs = jnp.stack([s1, s2])
    dsts = jnp.stack([d1, d2])

    deg = _deg_kernel(srcs, dsts)

    xpad = jnp.pad(jnp.stack([x1, x2]), ((0, 0), (0, NP - N), (0, 0)))
    dego4 = deg[:, :NP].reshape(2, NP // _RB, 1, _RB)
    degi4 = deg[:, NP:].reshape(2, NP // _RB, 1, _RB)
    xn3, ri4, ro4 = _prep_call(xpad, dego4, degi4)
    xn = xn3.reshape(2 * NP, D)
    ri = ri4.reshape(2 * NP, 1)
    ro = ro4.reshape(2 * NP, 1)

    wl = jnp.stack([W1, W2])
    bl = jnp.stack([b1.reshape(1, D), b2.reshape(1, D)])
    sc = jnp.stack([ro, jnp.ones_like(ro)])

    def layer(h, per):
        w, b_, s_ = per
        agg = _agg_kernel(h, srcs, dsts).reshape(2 * NP, D)
        return _mid_call(agg, ri, s_, w, b_), None

    h, _ = lax.scan(layer, xn, (wl, bl, sc))

    return _fin_call(h, graph_ids1.reshape(N, 1), graph_ids2.reshape(N, 1),
                     descriptors, C1W, C1b.reshape(1, 2 * D + 16),
                     C2W, C2b.reshape(1, D), C3W, C3b.reshape(1, D),
                     C4W, C4b.reshape(1, 1))


# speedup vs baseline: 1.8383x; 1.0433x over previous
import functools

import jax
import jax.numpy as jnp
from jax import lax
from jax.experimental import pallas as pl
from jax.experimental.pallas import tpu as pltpu
from jax.experimental.pallas import tpu_sc as plsc

N = 10000
E = 320000
D = 128
B = 64
NP = 10240
NT = 16
CH = 128
NB = 4
AGG_CHUNKS = 160
EP = NT * AGG_CHUNKS * CH
DEG_CHUNKS = 157
DP = NT * DEG_CHUNKS * CH

_mesh = plsc.VectorSubcoreMesh(core_axis_name="c", subcore_axis_name="s")


def _deg_body(src_hbm, dst_hbm, out_hbm, idx_v, idx2_v, ones_v, zeros_v,
              sdo_sh, sdi_sh, sem):
    c = lax.axis_index("c")
    t = lax.axis_index("s")

    def _fill(i, _):
        zeros_v[pl.ds(i * 16, 16)] = jnp.zeros((16,), jnp.float32)
        return 0

    lax.fori_loop(0, 80, _fill, 0)
    for j in range(8):
        ones_v[pl.ds(j * 16, 16)] = jnp.full((16,), 1.0, jnp.float32)
    pltpu.sync_copy(zeros_v, sdo_sh.at[pl.ds(t * 1280, 1280)])
    pltpu.sync_copy(zeros_v.at[pl.ds(0, 640)], sdi_sh.at[pl.ds(t * 640, 640)])
    pltpu.sync_copy(src_hbm.at[c, t], idx_v)
    pltpu.sync_copy(dst_hbm.at[c, t], idx2_v)
    plsc.subcore_barrier()

    def _scat(j, _):
        for u in range(8):
            pltpu.async_copy(ones_v, sdo_sh.at[idx_v.at[8 * j + u]], sem,
                             add=True)
        for u in range(8):
            pltpu.make_async_copy(ones_v, sdo_sh.at[idx_v.at[0]], sem).wait()
        for u in range(8):
            pltpu.async_copy(ones_v, sdi_sh.at[idx2_v.at[8 * j + u]], sem,
                             add=True)
        for u in range(8):
            pltpu.make_async_copy(ones_v, sdi_sh.at[idx2_v.at[0]], sem).wait()
        return 0

    lax.fori_loop(0, AGG_CHUNKS // 8, _scat, 0)
    plsc.subcore_barrier()

    @pl.when(t == 0)
    def _():
        pltpu.sync_copy(sdo_sh.at[pl.ds(c * NP, NP)],
                        out_hbm.at[c, pl.ds(0, NP)])

    @pl.when(t == 1)
    def _():
        pltpu.sync_copy(sdi_sh, out_hbm.at[c, pl.ds(NP, NP)])


_deg_kernel = pl.kernel(
    _deg_body,
    out_type=jax.ShapeDtypeStruct((2, 2 * NP), jnp.float32),
    mesh=_mesh,
    scratch_types=[
        pltpu.VMEM((AGG_CHUNKS, CH), jnp.int32),
        pltpu.VMEM((AGG_CHUNKS, CH), jnp.int32),
        pltpu.VMEM((CH,), jnp.float32),
        pltpu.VMEM((1280,), jnp.float32),
        pltpu.VMEM_SHARED((2 * NP,), jnp.float32),
        pltpu.VMEM_SHARED((NP,), jnp.float32),
        pltpu.SemaphoreType.DMA,
    ],
)


def _agg_body(xn_hbm, src_hbm, dst_hbm, out_hbm, src_v, dst_v, rows_v,
              zbuf_v, acc_sh, gsem):
    c = lax.axis_index("c")
    t = lax.axis_index("s")

    def _zrow(i, _):
        for j in range(D // 32):
            zbuf_v[i, pl.ds(j * 32, 32)] = jnp.zeros((32,), jnp.bfloat16)
        return 0

    lax.fori_loop(0, CH, _zrow, 0)
    pltpu.sync_copy(src_hbm.at[c, t], src_v)
    pltpu.sync_copy(dst_hbm.at[c, t], dst_v)
    for k in range(5):
        pltpu.sync_copy(zbuf_v, acc_sh.at[pl.ds(t * 640 + k * CH, CH)])
    plsc.subcore_barrier()

    for b in range(NB):
        pltpu.async_copy(xn_hbm.at[src_v.at[b]], rows_v.at[b], gsem.at[b])

    def _step(k, _):
        for b in range(NB):
            j = NB * k + b
            pltpu.make_async_copy(
                xn_hbm.at[src_v.at[j]], rows_v.at[b], gsem.at[b]).wait()
            pltpu.sync_copy(rows_v.at[b], acc_sh.at[dst_v.at[j]], add=True)

            @pl.when(k < AGG_CHUNKS // NB - 1)
            def _():
                pltpu.async_copy(
                    xn_hbm.at[src_v.at[NB * (k + 1) + b]], rows_v.at[b],
                    gsem.at[b])
        return 0

    lax.fori_loop(0, AGG_CHUNKS // NB, _step, 0)
    plsc.subcore_barrier()
    for k in range(5):
        pltpu.sync_copy(acc_sh.at[pl.ds(t * 640 + k * CH, CH)],
                        out_hbm.at[c, pl.ds(t * 640 + k * CH, CH)])


_agg_kernel = pl.kernel(
    _agg_body,
    out_type=jax.ShapeDtypeStruct((2, NP, D), jnp.bfloat16),
    mesh=_mesh,
    scratch_types=[
        pltpu.VMEM((AGG_CHUNKS, CH), jnp.int32),
        pltpu.VMEM((AGG_CHUNKS, CH), jnp.int32),
        pltpu.VMEM((NB, CH, D), jnp.bfloat16),
        pltpu.VMEM((CH, D), jnp.bfloat16),
        pltpu.VMEM_SHARED((NP, D), jnp.bfloat16),
        pltpu.SemaphoreType.DMA((NB,)),
    ],
    compiler_params=pltpu.CompilerParams(use_tc_tiling_on_sc=False),
)


_RB = 1024


def _prep_body(x_ref, dego_ref, degi_ref, xn_ref, ri_ref, ro_ref):
    ro = lax.rsqrt(jnp.maximum(dego_ref[0, 0, 0], 1.0))
    ri_ref[0, 0, 0] = lax.rsqrt(jnp.maximum(degi_ref[0, 0, 0], 1.0))
    ro_ref[0, 0, 0] = ro
    xn_ref[0] = (x_ref[0] * ro[:, None]).astype(jnp.bfloat16)


_prep_call = pl.pallas_call(
    _prep_body,
    grid=(2, NP // _RB),
    in_specs=[
        pl.BlockSpec((1, _RB, D), lambda g, i: (g, i, 0)),
        pl.BlockSpec((1, 1, 1, _RB), lambda g, i: (g, i, 0, 0)),
        pl.BlockSpec((1, 1, 1, _RB), lambda g, i: (g, i, 0, 0)),
    ],
    out_specs=[
        pl.BlockSpec((1, _RB, D), lambda g, i: (g, i, 0)),
        pl.BlockSpec((1, 1, 1, _RB), lambda g, i: (g, i, 0, 0)),
        pl.BlockSpec((1, 1, 1, _RB), lambda g, i: (g, i, 0, 0)),
    ],
    out_shape=[
        jax.ShapeDtypeStruct((2, NP, D), jnp.bfloat16),
        jax.ShapeDtypeStruct((2, NP // _RB, 1, _RB), jnp.float32),
        jax.ShapeDtypeStruct((2, NP // _RB, 1, _RB), jnp.float32),
    ],
)


def _mid_body(agg_ref, ri_ref, sc_ref, w_ref, b_ref, out_ref):
    a = agg_ref[...].astype(jnp.float32) * ri_ref[...]
    h = jnp.dot(a, w_ref[...], preferred_element_type=jnp.float32)
    h = jnp.maximum(h + b_ref[...], 0.0) * sc_ref[...]
    out_ref[...] = h.astype(jnp.bfloat16)


_mid_call = pl.pallas_call(
    _mid_body,
    grid=(2 * NP // _RB,),
    in_specs=[
        pl.BlockSpec((_RB, D), lambda i: (i, 0)),
        pl.BlockSpec((_RB, 1), lambda i: (i, 0)),
        pl.BlockSpec((_RB, 1), lambda i: (i, 0)),
        pl.BlockSpec((D, D), lambda i: (0, 0)),
        pl.BlockSpec((1, D), lambda i: (0, 0)),
    ],
    out_specs=pl.BlockSpec((_RB, D), lambda i: (i, 0)),
    out_shape=jax.ShapeDtypeStruct((2 * NP, D), jnp.bfloat16),
)


def _fin_body(h_ref, g1_ref, g2_ref, desc_ref,
              c1w_ref, c1b_ref, c2w_ref, c2b_ref, c3w_ref, c3b_ref,
              c4w_ref, c4b_ref, out_ref):
    iota = lax.broadcasted_iota(jnp.int32, (1, B), 1)

    def pool(g_ref, rows):
        m = (g_ref[...] == iota).astype(jnp.float32)
        s = lax.dot_general(m, rows, (((0,), (0,)), ((), ())),
                            preferred_element_type=jnp.float32)
        cnt = jnp.sum(m, axis=0)[:, None]
        return s / jnp.maximum(cnt, 1.0)

    hg1 = pool(g1_ref, h_ref[0:N].astype(jnp.float32))
    hg2 = pool(g2_ref, h_ref[NP:NP + N].astype(jnp.float32))

    c1w = c1w_ref[...]
    z = (jnp.dot(hg1, c1w[0:D], preferred_element_type=jnp.float32)
         + jnp.dot(hg2, c1w[D:2 * D], preferred_element_type=jnp.float32)
         + jnp.dot(desc_ref[...], c1w[2 * D:], preferred_element_type=jnp.float32)
         + c1b_ref[...])
    z = jnp.maximum(z, 0.0)
    z = jnp.maximum(jnp.dot(z, c2w_ref[...],
                            preferred_element_type=jnp.float32) + c2b_ref[...], 0.0)
    z = jnp.maximum(jnp.dot(z, c3w_ref[...],
                            preferred_element_type=jnp.float32) + c3b_ref[...], 0.0)
    out_ref[...] = jnp.dot(z, c4w_ref[...],
                           preferred_element_type=jnp.float32) + c4b_ref[...]


_fin_call = pl.pallas_call(
    _fin_body,
    out_shape=jax.ShapeDtypeStruct((B, 1), jnp.float32),
)


def _prep_agg_idx(ei, g):
    src, dst = ei[0], ei[1]
    padn = EP - E
    spread = N + (jnp.arange(padn, dtype=jnp.int32) % (NP - N))
    s = (jnp.concatenate([src, spread]) + g * NP).reshape(NT, AGG_CHUNKS, CH)
    d = jnp.concatenate([dst, spread]).reshape(NT, AGG_CHUNKS, CH)
    return s, d


def kernel(x1, x2, edge_index1, edge_index2, graph_ids1, graph_ids2,
           descriptors, W1, b1, W2, b2, C1W, C1b, C2W, C2b, C3W, C3b,
           C4W, C4b):
    s1, d1 = _prep_agg_idx(edge_index1, 0)
    s2, d2 = _prep_agg_idx(edge_index2, 1)
    srcs = jnp.stack([s1, s2])
    dsts = jnp.stack([d1, d2])

    deg = _deg_kernel(srcs, dsts)

    xpad = jnp.pad(jnp.stack([x1, x2]), ((0, 0), (0, NP - N), (0, 0)))
    dego4 = deg[:, :NP].reshape(2, NP // _RB, 1, _RB)
    degi4 = deg[:, NP:].reshape(2, NP // _RB, 1, _RB)
    xn3, ri4, ro4 = _prep_call(xpad, dego4, degi4)
    xn = xn3.reshape(2 * NP, D)
    ri = ri4.reshape(2 * NP, 1)
    ro = ro4.reshape(2 * NP, 1)

    wl = jnp.stack([W1, W2])
    bl = jnp.stack([b1.reshape(1, D), b2.reshape(1, D)])
    sc = jnp.stack([ro, jnp.ones_like(ro)])

    def layer(h, per):
        w, b_, s_ = per
        agg = _agg_kernel(h, srcs, dsts).reshape(2 * NP, D)
        return _mid_call(agg, ri, s_, w, b_), None

    h, _ = lax.scan(layer, xn, (wl, bl, sc))

    return _fin_call(h, graph_ids1.reshape(N, 1), graph_ids2.reshape(N, 1),
                     descriptors, C1W, C1b.reshape(1, 2 * D + 16),
                     C2W, C2b.reshape(1, D), C3W, C3b.reshape(1, D),
                     C4W, C4b.reshape(1, 1))


# speedup vs baseline: 1.8581x; 1.0108x over previous
import functools

import jax
import jax.numpy as jnp
from jax import lax
from jax.experimental import pallas as pl
from jax.experimental.pallas import tpu as pltpu
from jax.experimental.pallas import tpu_sc as plsc

N = 10000
E = 320000
D = 128
B = 64
NP = 10240
NT = 16
CH = 128
NB = 4
AGG_CHUNKS = 160
EP = NT * AGG_CHUNKS * CH
DEG_CHUNKS = 157
DP = NT * DEG_CHUNKS * CH

_mesh = plsc.VectorSubcoreMesh(core_axis_name="c", subcore_axis_name="s")


def _deg_body(src_hbm, dst_hbm, out_hbm, srco_hbm, dsto_hbm,
              idx_v, idx2_v, ones_v, zeros_v, sdo_sh, sdi_sh, sem):
    c = lax.axis_index("c")
    t = lax.axis_index("s")

    def _fill(i, _):
        zeros_v[pl.ds(i * 16, 16)] = jnp.zeros((16,), jnp.float32)
        return 0

    lax.fori_loop(0, 80, _fill, 0)
    for j in range(8):
        ones_v[pl.ds(j * 16, 16)] = jnp.full((16,), 1.0, jnp.float32)
    pltpu.sync_copy(zeros_v, sdo_sh.at[pl.ds(t * 1280, 1280)])
    pltpu.sync_copy(zeros_v.at[pl.ds(0, 640)], sdi_sh.at[pl.ds(t * 640, 640)])
    pltpu.sync_copy(src_hbm.at[c, t], idx_v)
    pltpu.sync_copy(dst_hbm.at[c, t], idx2_v)
    pltpu.sync_copy(idx_v, srco_hbm.at[c, t])
    pltpu.sync_copy(idx2_v, dsto_hbm.at[c, t])
    plsc.subcore_barrier()

    def _scat(j, _):
        for u in range(8):
            pltpu.async_copy(ones_v, sdo_sh.at[idx_v.at[8 * j + u]], sem,
                             add=True)
        for u in range(8):
            pltpu.make_async_copy(ones_v, sdo_sh.at[idx_v.at[0]], sem).wait()
        for u in range(8):
            pltpu.async_copy(ones_v, sdi_sh.at[idx2_v.at[8 * j + u]], sem,
                             add=True)
        for u in range(8):
            pltpu.make_async_copy(ones_v, sdi_sh.at[idx2_v.at[0]], sem).wait()
        return 0

    lax.fori_loop(0, AGG_CHUNKS // 8, _scat, 0)
    plsc.subcore_barrier()

    @pl.when(t == 0)
    def _():
        pltpu.sync_copy(sdo_sh.at[pl.ds(c * NP, NP)],
                        out_hbm.at[c, pl.ds(0, NP)])

    @pl.when(t == 1)
    def _():
        pltpu.sync_copy(sdi_sh, out_hbm.at[c, pl.ds(NP, NP)])


_deg_kernel = pl.kernel(
    _deg_body,
    out_type=[
        jax.ShapeDtypeStruct((2, 2 * NP), jnp.float32),
        jax.ShapeDtypeStruct((2, NT, AGG_CHUNKS, CH), jnp.int32),
        jax.ShapeDtypeStruct((2, NT, AGG_CHUNKS, CH), jnp.int32),
    ],
    mesh=_mesh,
    scratch_types=[
        pltpu.VMEM((AGG_CHUNKS, CH), jnp.int32),
        pltpu.VMEM((AGG_CHUNKS, CH), jnp.int32),
        pltpu.VMEM((CH,), jnp.float32),
        pltpu.VMEM((1280,), jnp.float32),
        pltpu.VMEM_SHARED((2 * NP,), jnp.float32),
        pltpu.VMEM_SHARED((NP,), jnp.float32),
        pltpu.SemaphoreType.DMA,
    ],
)


def _agg_body(xn_hbm, src_hbm, dst_hbm, out_hbm, src_v, dst_v, rows_v,
              zbuf_v, acc_sh, gsem):
    c = lax.axis_index("c")
    t = lax.axis_index("s")

    def _zrow(i, _):
        for j in range(D // 32):
            zbuf_v[i, pl.ds(j * 32, 32)] = jnp.zeros((32,), jnp.bfloat16)
        return 0

    lax.fori_loop(0, CH, _zrow, 0)
    pltpu.sync_copy(src_hbm.at[c, t], src_v)
    pltpu.sync_copy(dst_hbm.at[c, t], dst_v)
    for k in range(5):
        pltpu.sync_copy(zbuf_v, acc_sh.at[pl.ds(t * 640 + k * CH, CH)])
    plsc.subcore_barrier()

    for b in range(NB):
        pltpu.async_copy(xn_hbm.at[src_v.at[b]], rows_v.at[b], gsem.at[b])

    def _step(k, _):
        for b in range(NB):
            j = NB * k + b
            pltpu.make_async_copy(
                xn_hbm.at[src_v.at[j]], rows_v.at[b], gsem.at[b]).wait()
            pltpu.sync_copy(rows_v.at[b], acc_sh.at[dst_v.at[j]], add=True)

            @pl.when(k < AGG_CHUNKS // NB - 1)
            def _():
                pltpu.async_copy(
                    xn_hbm.at[src_v.at[NB * (k + 1) + b]], rows_v.at[b],
                    gsem.at[b])
        return 0

    lax.fori_loop(0, AGG_CHUNKS // NB, _step, 0)
    plsc.subcore_barrier()
    for k in range(5):
        pltpu.sync_copy(acc_sh.at[pl.ds(t * 640 + k * CH, CH)],
                        out_hbm.at[c, pl.ds(t * 640 + k * CH, CH)])


_agg_kernel = pl.kernel(
    _agg_body,
    out_type=jax.ShapeDtypeStruct((2, NP, D), jnp.bfloat16),
    mesh=_mesh,
    scratch_types=[
        pltpu.VMEM((AGG_CHUNKS, CH), jnp.int32),
        pltpu.VMEM((AGG_CHUNKS, CH), jnp.int32),
        pltpu.VMEM((NB, CH, D), jnp.bfloat16),
        pltpu.VMEM((CH, D), jnp.bfloat16),
        pltpu.VMEM_SHARED((NP, D), jnp.bfloat16),
        pltpu.SemaphoreType.DMA((NB,)),
    ],
    compiler_params=pltpu.CompilerParams(use_tc_tiling_on_sc=False),
)


_RB = 1024


def _prep_body(x_ref, dego_ref, degi_ref, xn_ref, ri_ref, ro_ref):
    ro = lax.rsqrt(jnp.maximum(dego_ref[0, 0, 0], 1.0))
    ri_ref[0, 0, 0] = lax.rsqrt(jnp.maximum(degi_ref[0, 0, 0], 1.0))
    ro_ref[0, 0, 0] = ro
    xn = (x_ref[0] * ro[:, None]).astype(jnp.bfloat16)
    xn_ref[...] = xn.reshape(_RB * D)


_prep_call = pl.pallas_call(
    _prep_body,
    grid=(2, NP // _RB),
    in_specs=[
        pl.BlockSpec((1, _RB, D), lambda g, i: (g, i, 0)),
        pl.BlockSpec((1, 1, 1, _RB), lambda g, i: (g, i, 0, 0)),
        pl.BlockSpec((1, 1, 1, _RB), lambda g, i: (g, i, 0, 0)),
    ],
    out_specs=[
        pl.BlockSpec((_RB * D,), lambda g, i: (g * (NP // _RB) + i,)),
        pl.BlockSpec((1, 1, 1, _RB), lambda g, i: (g, i, 0, 0)),
        pl.BlockSpec((1, 1, 1, _RB), lambda g, i: (g, i, 0, 0)),
    ],
    out_shape=[
        jax.ShapeDtypeStruct((2 * NP * D,), jnp.bfloat16),
        jax.ShapeDtypeStruct((2, NP // _RB, 1, _RB), jnp.float32),
        jax.ShapeDtypeStruct((2, NP // _RB, 1, _RB), jnp.float32),
    ],
)


def _mid_body(agg_ref, ri_ref, sc_ref, w_ref, b_ref, out_ref):
    a = agg_ref[...].reshape(_RB, D).astype(jnp.float32) * ri_ref[...]
    h = jnp.dot(a, w_ref[...], preferred_element_type=jnp.float32)
    h = jnp.maximum(h + b_ref[...], 0.0) * sc_ref[...]
    out_ref[...] = h.astype(jnp.bfloat16).reshape(_RB * D)


_mid_call = pl.pallas_call(
    _mid_body,
    grid=(2 * NP // _RB,),
    in_specs=[
        pl.BlockSpec((_RB * D,), lambda i: (i,)),
        pl.BlockSpec((_RB, 1), lambda i: (i, 0)),
        pl.BlockSpec((_RB, 1), lambda i: (i, 0)),
        pl.BlockSpec((D, D), lambda i: (0, 0)),
        pl.BlockSpec((1, D), lambda i: (0, 0)),
    ],
    out_specs=pl.BlockSpec((_RB * D,), lambda i: (i,)),
    out_shape=jax.ShapeDtypeStruct((2 * NP * D,), jnp.bfloat16),
)


def _fin_body(h_ref, g1_ref, g2_ref, desc_ref,
              c1w_ref, c1b_ref, c2w_ref, c2b_ref, c3w_ref, c3b_ref,
              c4w_ref, c4b_ref, out_ref):
    h2d = h_ref[...].reshape(2 * NP, D)
    iota = lax.broadcasted_iota(jnp.int32, (1, B), 1)

    def pool(g_ref, rows):
        m = (g_ref[...] == iota).astype(jnp.float32)
        s = lax.dot_general(m, rows, (((0,), (0,)), ((), ())),
                            preferred_element_type=jnp.float32)
        cnt = jnp.sum(m, axis=0)[:, None]
        return s / jnp.maximum(cnt, 1.0)

    hg1 = pool(g1_ref, h2d[0:N].astype(jnp.float32))
    hg2 = pool(g2_ref, h2d[NP:NP + N].astype(jnp.float32))

    c1w = c1w_ref[...]
    z = (jnp.dot(hg1, c1w[0:D], preferred_element_type=jnp.float32)
         + jnp.dot(hg2, c1w[D:2 * D], preferred_element_type=jnp.float32)
         + jnp.dot(desc_ref[...], c1w[2 * D:], preferred_element_type=jnp.float32)
         + c1b_ref[...])
    z = jnp.maximum(z, 0.0)
    z = jnp.maximum(jnp.dot(z, c2w_ref[...],
                            preferred_element_type=jnp.float32) + c2b_ref[...], 0.0)
    z = jnp.maximum(jnp.dot(z, c3w_ref[...],
                            preferred_element_type=jnp.float32) + c3b_ref[...], 0.0)
    out_ref[...] = jnp.dot(z, c4w_ref[...],
                           preferred_element_type=jnp.float32) + c4b_ref[...]


_fin_call = pl.pallas_call(
    _fin_body,
    out_shape=jax.ShapeDtypeStruct((B, 1), jnp.float32),
)


def _prep_agg_idx(ei, g):
    src, dst = ei[0], ei[1]
    padn = EP - E
    spread = N + (jnp.arange(padn, dtype=jnp.int32) % (NP - N))
    s = (jnp.concatenate([src, spread]) + g * NP).reshape(NT, AGG_CHUNKS, CH)
    d = jnp.concatenate([dst, spread]).reshape(NT, AGG_CHUNKS, CH)
    return s, d


def kernel(x1, x2, edge_index1, edge_index2, graph_ids1, graph_ids2,
           descriptors, W1, b1, W2, b2, C1W, C1b, C2W, C2b, C3W, C3b,
           C4W, C4b):
    s1, d1 = _prep_agg_idx(edge_index1, 0)
    s2, d2 = _prep_agg_idx(edge_index2, 1)
    srcs = jnp.stack([s1, s2])
    dsts = jnp.stack([d1, d2])

    deg, srcs, dsts = _deg_kernel(srcs, dsts)

    xpad = jnp.pad(jnp.stack([x1, x2]), ((0, 0), (0, NP - N), (0, 0)))
    dego4 = deg[:, :NP].reshape(2, NP // _RB, 1, _RB)
    degi4 = deg[:, NP:].reshape(2, NP // _RB, 1, _RB)
    xn, ri4, ro4 = _prep_call(xpad, dego4, degi4)
    ri = ri4.reshape(2 * NP, 1)
    ro = ro4.reshape(2 * NP, 1)

    wl = jnp.stack([W1, W2])
    bl = jnp.stack([b1.reshape(1, D), b2.reshape(1, D)])
    sc = jnp.stack([ro, jnp.ones_like(ro)])

    def layer(h, per):
        w, b_, s_ = per
        agg = _agg_kernel(h.reshape(2 * NP, D), srcs, dsts)
        return _mid_call(agg.reshape(2 * NP * D), ri, s_, w, b_), None

    h, _ = lax.scan(layer, xn, (wl, bl, sc))

    return _fin_call(h, graph_ids1.reshape(N, 1), graph_ids2.reshape(N, 1),
                     descriptors, C1W, C1b.reshape(1, 2 * D + 16),
                     C2W, C2b.reshape(1, D), C3W, C3b.reshape(1, D),
                     C4W, C4b.reshape(1, 1))


# speedup vs baseline: 1.8719x; 1.0074x over previous
import functools

import jax
import jax.numpy as jnp
from jax import lax
from jax.experimental import pallas as pl
from jax.experimental.pallas import tpu as pltpu
from jax.experimental.pallas import tpu_sc as plsc

N = 10000
E = 320000
D = 128
B = 64
NP = 10240
NT = 16
CH = 128
NB = 4
AGG_CHUNKS = 160
EP = NT * AGG_CHUNKS * CH
DEG_CHUNKS = 157
DP = NT * DEG_CHUNKS * CH

_mesh = plsc.VectorSubcoreMesh(core_axis_name="c", subcore_axis_name="s")


_RAW_CHUNKS = E // CH


def _deg_body(e1_hbm, e2_hbm, out_hbm, srco_hbm, dsto_hbm,
              idx_v, idx2_v, ones_v, zeros_v, sdo_sh, sdi_sh, sem):
    c = lax.axis_index("c")
    t = lax.axis_index("s")

    def _fill(i, _):
        zeros_v[pl.ds(i * 16, 16)] = jnp.zeros((16,), jnp.float32)
        return 0

    lax.fori_loop(0, 80, _fill, 0)
    for j in range(8):
        ones_v[pl.ds(j * 16, 16)] = jnp.full((16,), 1.0, jnp.float32)
    pltpu.sync_copy(zeros_v, sdo_sh.at[pl.ds(t * 1280, 1280)])
    pltpu.sync_copy(zeros_v.at[pl.ds(0, 640)], sdi_sh.at[pl.ds(t * 640, 640)])

    base = t * 156 + jnp.minimum(t, 4)
    iota = lax.iota(jnp.int32, 16)
    for r in range(156, AGG_CHUNKS):
        for j in range(CH // 16):
            v = N + ((r * CH + j * 16 + iota) % (NP - N))
            idx_v[r, pl.ds(j * 16, 16)] = v
            idx2_v[r, pl.ds(j * 16, 16)] = v

    @pl.when(c == 0)
    def _():
        pltpu.sync_copy(e1_hbm.at[0, pl.ds(base, 156)], idx_v.at[pl.ds(0, 156)])
        pltpu.sync_copy(e1_hbm.at[1, pl.ds(base, 156)], idx2_v.at[pl.ds(0, 156)])

    @pl.when(c == 1)
    def _():
        pltpu.sync_copy(e2_hbm.at[0, pl.ds(base, 156)], idx_v.at[pl.ds(0, 156)])
        pltpu.sync_copy(e2_hbm.at[1, pl.ds(base, 156)], idx2_v.at[pl.ds(0, 156)])

    @pl.when((c == 0) & (t < 4))
    def _():
        pltpu.sync_copy(e1_hbm.at[0, pl.ds(base + 156, 1)],
                        idx_v.at[pl.ds(156, 1)])
        pltpu.sync_copy(e1_hbm.at[1, pl.ds(base + 156, 1)],
                        idx2_v.at[pl.ds(156, 1)])

    @pl.when((c == 1) & (t < 4))
    def _():
        pltpu.sync_copy(e2_hbm.at[0, pl.ds(base + 156, 1)],
                        idx_v.at[pl.ds(156, 1)])
        pltpu.sync_copy(e2_hbm.at[1, pl.ds(base + 156, 1)],
                        idx2_v.at[pl.ds(156, 1)])

    off = c * NP

    def _add_off(i, _):
        for j in range(CH // 16):
            sl = pl.ds(j * 16, 16)
            idx_v[i, sl] = idx_v[i, sl] + off
        return 0

    lax.fori_loop(0, AGG_CHUNKS, _add_off, 0)
    pltpu.sync_copy(idx_v, srco_hbm.at[c, t])
    pltpu.sync_copy(idx2_v, dsto_hbm.at[c, t])
    plsc.subcore_barrier()

    def _scat(j, _):
        for u in range(8):
            pltpu.async_copy(ones_v, sdo_sh.at[idx_v.at[8 * j + u]], sem,
                             add=True)
        for u in range(8):
            pltpu.make_async_copy(ones_v, sdo_sh.at[idx_v.at[0]], sem).wait()
        for u in range(8):
            pltpu.async_copy(ones_v, sdi_sh.at[idx2_v.at[8 * j + u]], sem,
                             add=True)
        for u in range(8):
            pltpu.make_async_copy(ones_v, sdi_sh.at[idx2_v.at[0]], sem).wait()
        return 0

    lax.fori_loop(0, AGG_CHUNKS // 8, _scat, 0)
    plsc.subcore_barrier()

    @pl.when(t == 0)
    def _():
        pltpu.sync_copy(sdo_sh.at[pl.ds(c * NP, NP)],
                        out_hbm.at[c, pl.ds(0, NP)])

    @pl.when(t == 1)
    def _():
        pltpu.sync_copy(sdi_sh, out_hbm.at[c, pl.ds(NP, NP)])


_deg_kernel = pl.kernel(
    _deg_body,
    out_type=[
        jax.ShapeDtypeStruct((2, 2 * NP), jnp.float32),
        jax.ShapeDtypeStruct((2, NT, AGG_CHUNKS, CH), jnp.int32),
        jax.ShapeDtypeStruct((2, NT, AGG_CHUNKS, CH), jnp.int32),
    ],
    mesh=_mesh,
    scratch_types=[
        pltpu.VMEM((AGG_CHUNKS, CH), jnp.int32),
        pltpu.VMEM((AGG_CHUNKS, CH), jnp.int32),
        pltpu.VMEM((CH,), jnp.float32),
        pltpu.VMEM((1280,), jnp.float32),
        pltpu.VMEM_SHARED((2 * NP,), jnp.float32),
        pltpu.VMEM_SHARED((NP,), jnp.float32),
        pltpu.SemaphoreType.DMA,
    ],
    compiler_params=pltpu.CompilerParams(use_tc_tiling_on_sc=False),
)


def _agg_body(xn_hbm, src_hbm, dst_hbm, out_hbm, src_v, dst_v, rows_v,
              zbuf_v, acc_sh, gsem):
    c = lax.axis_index("c")
    t = lax.axis_index("s")

    def _zrow(i, _):
        for j in range(D // 32):
            zbuf_v[i, pl.ds(j * 32, 32)] = jnp.zeros((32,), jnp.bfloat16)
        return 0

    lax.fori_loop(0, CH, _zrow, 0)
    pltpu.sync_copy(src_hbm.at[c, t], src_v)
    pltpu.sync_copy(dst_hbm.at[c, t], dst_v)
    for k in range(5):
        pltpu.sync_copy(zbuf_v, acc_sh.at[pl.ds(t * 640 + k * CH, CH)])
    plsc.subcore_barrier()

    for b in range(NB):
        pltpu.async_copy(xn_hbm.at[src_v.at[b]], rows_v.at[b], gsem.at[b])

    def _step(k, _):
        for b in range(NB):
            j = NB * k + b
            pltpu.make_async_copy(
                xn_hbm.at[src_v.at[j]], rows_v.at[b], gsem.at[b]).wait()
            pltpu.sync_copy(rows_v.at[b], acc_sh.at[dst_v.at[j]], add=True)

            @pl.when(k < AGG_CHUNKS // NB - 1)
            def _():
                pltpu.async_copy(
                    xn_hbm.at[src_v.at[NB * (k + 1) + b]], rows_v.at[b],
                    gsem.at[b])
        return 0

    lax.fori_loop(0, AGG_CHUNKS // NB, _step, 0)
    plsc.subcore_barrier()
    for k in range(5):
        pltpu.sync_copy(acc_sh.at[pl.ds(t * 640 + k * CH, CH)],
                        out_hbm.at[c, pl.ds(t * 640 + k * CH, CH)])


_agg_kernel = pl.kernel(
    _agg_body,
    out_type=jax.ShapeDtypeStruct((2, NP, D), jnp.bfloat16),
    mesh=_mesh,
    scratch_types=[
        pltpu.VMEM((AGG_CHUNKS, CH), jnp.int32),
        pltpu.VMEM((AGG_CHUNKS, CH), jnp.int32),
        pltpu.VMEM((NB, CH, D), jnp.bfloat16),
        pltpu.VMEM((CH, D), jnp.bfloat16),
        pltpu.VMEM_SHARED((NP, D), jnp.bfloat16),
        pltpu.SemaphoreType.DMA((NB,)),
    ],
    compiler_params=pltpu.CompilerParams(use_tc_tiling_on_sc=False),
)


_RB = 1024


def _prep_body(x_ref, dego_ref, degi_ref, xn_ref, ri_ref, ro_ref):
    ro = lax.rsqrt(jnp.maximum(dego_ref[0, 0, 0], 1.0))
    ri_ref[...] = lax.rsqrt(jnp.maximum(degi_ref[0, 0, 0], 1.0))[:, None]
    ro_ref[...] = ro[:, None]
    xn = (x_ref[0] * ro[:, None]).astype(jnp.bfloat16)
    xn_ref[...] = xn.reshape(_RB * D)


_prep_call = pl.pallas_call(
    _prep_body,
    grid=(2, NP // _RB),
    in_specs=[
        pl.BlockSpec((1, _RB, D), lambda g, i: (g, i, 0)),
        pl.BlockSpec((1, 1, 1, _RB), lambda g, i: (g, i, 0, 0)),
        pl.BlockSpec((1, 1, 1, _RB), lambda g, i: (g, i, 0, 0)),
    ],
    out_specs=[
        pl.BlockSpec((_RB * D,), lambda g, i: (g * (NP // _RB) + i,)),
        pl.BlockSpec((_RB, 1), lambda g, i: (g * (NP // _RB) + i, 0)),
        pl.BlockSpec((_RB, 1), lambda g, i: (g * (NP // _RB) + i, 0)),
    ],
    out_shape=[
        jax.ShapeDtypeStruct((2 * NP * D,), jnp.bfloat16),
        jax.ShapeDtypeStruct((2 * NP, 1), jnp.float32),
        jax.ShapeDtypeStruct((2 * NP, 1), jnp.float32),
    ],
)


def _mid_body(agg_ref, ri_ref, sc_ref, w_ref, b_ref, out_ref):
    a = agg_ref[...].reshape(_RB, D).astype(jnp.float32) * ri_ref[...]
    h = jnp.dot(a, w_ref[...], preferred_element_type=jnp.float32)
    h = jnp.maximum(h + b_ref[...], 0.0) * sc_ref[...]
    out_ref[...] = h.astype(jnp.bfloat16).reshape(_RB * D)


_mid_call = pl.pallas_call(
    _mid_body,
    grid=(2 * NP // _RB,),
    in_specs=[
        pl.BlockSpec((_RB * D,), lambda i: (i,)),
        pl.BlockSpec((_RB, 1), lambda i: (i, 0)),
        pl.BlockSpec((_RB, 1), lambda i: (i, 0)),
        pl.BlockSpec((D, D), lambda i: (0, 0)),
        pl.BlockSpec((1, D), lambda i: (0, 0)),
    ],
    out_specs=pl.BlockSpec((_RB * D,), lambda i: (i,)),
    out_shape=jax.ShapeDtypeStruct((2 * NP * D,), jnp.bfloat16),
)


def _fin_body(h_ref, g1_ref, g2_ref, desc_ref,
              c1w_ref, c1b_ref, c2w_ref, c2b_ref, c3w_ref, c3b_ref,
              c4w_ref, c4b_ref, out_ref):
    h2d = h_ref[...].reshape(2 * NP, D)
    iota = lax.broadcasted_iota(jnp.int32, (1, B), 1)

    def pool(g_ref, rows):
        m = (g_ref[...] == iota).astype(jnp.float32)
        s = lax.dot_general(m, rows, (((0,), (0,)), ((), ())),
                            preferred_element_type=jnp.float32)
        cnt = jnp.sum(m, axis=0)[:, None]
        return s / jnp.maximum(cnt, 1.0)

    hg1 = pool(g1_ref, h2d[0:N].astype(jnp.float32))
    hg2 = pool(g2_ref, h2d[NP:NP + N].astype(jnp.float32))

    c1w = c1w_ref[...]
    z = (jnp.dot(hg1, c1w[0:D], preferred_element_type=jnp.float32)
         + jnp.dot(hg2, c1w[D:2 * D], preferred_element_type=jnp.float32)
         + jnp.dot(desc_ref[...], c1w[2 * D:], preferred_element_type=jnp.float32)
         + c1b_ref[...])
    z = jnp.maximum(z, 0.0)
    z = jnp.maximum(jnp.dot(z, c2w_ref[...],
                            preferred_element_type=jnp.float32) + c2b_ref[...], 0.0)
    z = jnp.maximum(jnp.dot(z, c3w_ref[...],
                            preferred_element_type=jnp.float32) + c3b_ref[...], 0.0)
    out_ref[...] = jnp.dot(z, c4w_ref[...],
                           preferred_element_type=jnp.float32) + c4b_ref[...]


_fin_call = pl.pallas_call(
    _fin_body,
    out_shape=jax.ShapeDtypeStruct((B, 1), jnp.float32),
)


def kernel(x1, x2, edge_index1, edge_index2, graph_ids1, graph_ids2,
           descriptors, W1, b1, W2, b2, C1W, C1b, C2W, C2b, C3W, C3b,
           C4W, C4b):
    e1r = edge_index1.reshape(2, _RAW_CHUNKS, CH)
    e2r = edge_index2.reshape(2, _RAW_CHUNKS, CH)
    deg, srcs, dsts = _deg_kernel(e1r, e2r)

    xpad = jnp.pad(jnp.stack([x1, x2]), ((0, 0), (0, NP - N), (0, 0)))
    dego4 = deg[:, :NP].reshape(2, NP // _RB, 1, _RB)
    degi4 = deg[:, NP:].reshape(2, NP // _RB, 1, _RB)
    xn, ri, ro = _prep_call(xpad, dego4, degi4)

    wl = jnp.stack([W1, W2])
    bl = jnp.stack([b1.reshape(1, D), b2.reshape(1, D)])
    sc = jnp.stack([ro, jnp.ones_like(ro)])

    def layer(h, per):
        w, b_, s_ = per
        agg = _agg_kernel(h.reshape(2 * NP, D), srcs, dsts)
        return _mid_call(agg.reshape(2 * NP * D), ri, s_, w, b_), None

    h, _ = lax.scan(layer, xn, (wl, bl, sc))

    return _fin_call(h, graph_ids1.reshape(N, 1), graph_ids2.reshape(N, 1),
                     descriptors, C1W, C1b.reshape(1, 2 * D + 16),
                     C2W, C2b.reshape(1, D), C3W, C3b.reshape(1, D),
                     C4W, C4b.reshape(1, 1))


# speedup vs baseline: 1.8719x; 1.0000x over previous
import jax
import jax.numpy as jnp
from jax import lax
from jax.experimental import pallas as pl
from jax.experimental.pallas import tpu as pltpu
from jax.experimental.pallas import tpu_sc as plsc

N = 10000
E = 320000
D = 128
B = 64
NP = 10240
NT = 16
CH = 128
NB = 4
AGG_CHUNKS = 160

_mesh = plsc.VectorSubcoreMesh(core_axis_name="c", subcore_axis_name="s")


_RAW_CHUNKS = E // CH


def _deg_body(e1_hbm, e2_hbm, out_hbm, srco_hbm, dsto_hbm,
              idx_v, idx2_v, ones_v, zeros_v, sdo_sh, sdi_sh, sem):
    c = lax.axis_index("c")
    t = lax.axis_index("s")

    def _fill(i, _):
        zeros_v[pl.ds(i * 16, 16)] = jnp.zeros((16,), jnp.float32)
        return 0

    lax.fori_loop(0, 80, _fill, 0)
    for j in range(8):
        ones_v[pl.ds(j * 16, 16)] = jnp.full((16,), 1.0, jnp.float32)
    pltpu.sync_copy(zeros_v, sdo_sh.at[pl.ds(t * 1280, 1280)])
    pltpu.sync_copy(zeros_v.at[pl.ds(0, 640)], sdi_sh.at[pl.ds(t * 640, 640)])

    base = t * 156 + jnp.minimum(t, 4)
    iota = lax.iota(jnp.int32, 16)
    for r in range(156, AGG_CHUNKS):
        for j in range(CH // 16):
            v = N + ((r * CH + j * 16 + iota) % (NP - N))
            idx_v[r, pl.ds(j * 16, 16)] = v
            idx2_v[r, pl.ds(j * 16, 16)] = v

    @pl.when(c == 0)
    def _():
        pltpu.sync_copy(e1_hbm.at[0, pl.ds(base, 156)], idx_v.at[pl.ds(0, 156)])
        pltpu.sync_copy(e1_hbm.at[1, pl.ds(base, 156)], idx2_v.at[pl.ds(0, 156)])

    @pl.when(c == 1)
    def _():
        pltpu.sync_copy(e2_hbm.at[0, pl.ds(base, 156)], idx_v.at[pl.ds(0, 156)])
        pltpu.sync_copy(e2_hbm.at[1, pl.ds(base, 156)], idx2_v.at[pl.ds(0, 156)])

    @pl.when((c == 0) & (t < 4))
    def _():
        pltpu.sync_copy(e1_hbm.at[0, pl.ds(base + 156, 1)],
                        idx_v.at[pl.ds(156, 1)])
        pltpu.sync_copy(e1_hbm.at[1, pl.ds(base + 156, 1)],
                        idx2_v.at[pl.ds(156, 1)])

    @pl.when((c == 1) & (t < 4))
    def _():
        pltpu.sync_copy(e2_hbm.at[0, pl.ds(base + 156, 1)],
                        idx_v.at[pl.ds(156, 1)])
        pltpu.sync_copy(e2_hbm.at[1, pl.ds(base + 156, 1)],
                        idx2_v.at[pl.ds(156, 1)])

    off = c * NP

    def _add_off(i, _):
        for j in range(CH // 16):
            sl = pl.ds(j * 16, 16)
            idx_v[i, sl] = idx_v[i, sl] + off
        return 0

    lax.fori_loop(0, AGG_CHUNKS, _add_off, 0)
    pltpu.sync_copy(idx_v, srco_hbm.at[c, t])
    pltpu.sync_copy(idx2_v, dsto_hbm.at[c, t])
    plsc.subcore_barrier()

    def _scat(j, _):
        for u in range(8):
            pltpu.async_copy(ones_v, sdo_sh.at[idx_v.at[8 * j + u]], sem,
                             add=True)
        for u in range(8):
            pltpu.make_async_copy(ones_v, sdo_sh.at[idx_v.at[0]], sem).wait()
        for u in range(8):
            pltpu.async_copy(ones_v, sdi_sh.at[idx2_v.at[8 * j + u]], sem,
                             add=True)
        for u in range(8):
            pltpu.make_async_copy(ones_v, sdi_sh.at[idx2_v.at[0]], sem).wait()
        return 0

    lax.fori_loop(0, AGG_CHUNKS // 8, _scat, 0)
    plsc.subcore_barrier()

    @pl.when(t == 0)
    def _():
        pltpu.sync_copy(sdo_sh.at[pl.ds(c * NP, NP)],
                        out_hbm.at[c, pl.ds(0, NP)])

    @pl.when(t == 1)
    def _():
        pltpu.sync_copy(sdi_sh, out_hbm.at[c, pl.ds(NP, NP)])


_deg_kernel = pl.kernel(
    _deg_body,
    out_type=[
        jax.ShapeDtypeStruct((2, 2 * NP), jnp.float32),
        jax.ShapeDtypeStruct((2, NT, AGG_CHUNKS, CH), jnp.int32),
        jax.ShapeDtypeStruct((2, NT, AGG_CHUNKS, CH), jnp.int32),
    ],
    mesh=_mesh,
    scratch_types=[
        pltpu.VMEM((AGG_CHUNKS, CH), jnp.int32),
        pltpu.VMEM((AGG_CHUNKS, CH), jnp.int32),
        pltpu.VMEM((CH,), jnp.float32),
        pltpu.VMEM((1280,), jnp.float32),
        pltpu.VMEM_SHARED((2 * NP,), jnp.float32),
        pltpu.VMEM_SHARED((NP,), jnp.float32),
        pltpu.SemaphoreType.DMA,
    ],
    compiler_params=pltpu.CompilerParams(use_tc_tiling_on_sc=False),
)


def _agg_body(xn_hbm, src_hbm, dst_hbm, out_hbm, src_v, dst_v, rows_v,
              zbuf_v, acc_sh, gsem):
    c = lax.axis_index("c")
    t = lax.axis_index("s")

    def _zrow(i, _):
        for j in range(D // 32):
            zbuf_v[i, pl.ds(j * 32, 32)] = jnp.zeros((32,), jnp.bfloat16)
        return 0

    lax.fori_loop(0, CH, _zrow, 0)
    pltpu.sync_copy(src_hbm.at[c, t], src_v)
    pltpu.sync_copy(dst_hbm.at[c, t], dst_v)
    for k in range(5):
        pltpu.sync_copy(zbuf_v, acc_sh.at[pl.ds(t * 640 + k * CH, CH)])
    plsc.subcore_barrier()

    for b in range(NB):
        pltpu.async_copy(xn_hbm.at[src_v.at[b]], rows_v.at[b], gsem.at[b])

    def _step(k, _):
        for b in range(NB):
            j = NB * k + b
            pltpu.make_async_copy(
                xn_hbm.at[src_v.at[j]], rows_v.at[b], gsem.at[b]).wait()
            pltpu.sync_copy(rows_v.at[b], acc_sh.at[dst_v.at[j]], add=True)

            @pl.when(k < AGG_CHUNKS // NB - 1)
            def _():
                pltpu.async_copy(
                    xn_hbm.at[src_v.at[NB * (k + 1) + b]], rows_v.at[b],
                    gsem.at[b])
        return 0

    lax.fori_loop(0, AGG_CHUNKS // NB, _step, 0)
    plsc.subcore_barrier()
    for k in range(5):
        pltpu.sync_copy(acc_sh.at[pl.ds(t * 640 + k * CH, CH)],
                        out_hbm.at[c, pl.ds(t * 640 + k * CH, CH)])


_agg_kernel = pl.kernel(
    _agg_body,
    out_type=jax.ShapeDtypeStruct((2, NP, D), jnp.bfloat16),
    mesh=_mesh,
    scratch_types=[
        pltpu.VMEM((AGG_CHUNKS, CH), jnp.int32),
        pltpu.VMEM((AGG_CHUNKS, CH), jnp.int32),
        pltpu.VMEM((NB, CH, D), jnp.bfloat16),
        pltpu.VMEM((CH, D), jnp.bfloat16),
        pltpu.VMEM_SHARED((NP, D), jnp.bfloat16),
        pltpu.SemaphoreType.DMA((NB,)),
    ],
    compiler_params=pltpu.CompilerParams(use_tc_tiling_on_sc=False),
)


_RB = 1024


def _prep_body(x_ref, dego_ref, degi_ref, xn_ref, ri_ref, ro_ref):
    ro = lax.rsqrt(jnp.maximum(dego_ref[0, 0, 0], 1.0))
    ri_ref[...] = lax.rsqrt(jnp.maximum(degi_ref[0, 0, 0], 1.0))[:, None]
    ro_ref[...] = ro[:, None]
    xn = (x_ref[0] * ro[:, None]).astype(jnp.bfloat16)
    xn_ref[...] = xn.reshape(_RB * D)


_prep_call = pl.pallas_call(
    _prep_body,
    grid=(2, NP // _RB),
    in_specs=[
        pl.BlockSpec((1, _RB, D), lambda g, i: (g, i, 0)),
        pl.BlockSpec((1, 1, 1, _RB), lambda g, i: (g, i, 0, 0)),
        pl.BlockSpec((1, 1, 1, _RB), lambda g, i: (g, i, 0, 0)),
    ],
    out_specs=[
        pl.BlockSpec((_RB * D,), lambda g, i: (g * (NP // _RB) + i,)),
        pl.BlockSpec((_RB, 1), lambda g, i: (g * (NP // _RB) + i, 0)),
        pl.BlockSpec((_RB, 1), lambda g, i: (g * (NP // _RB) + i, 0)),
    ],
    out_shape=[
        jax.ShapeDtypeStruct((2 * NP * D,), jnp.bfloat16),
        jax.ShapeDtypeStruct((2 * NP, 1), jnp.float32),
        jax.ShapeDtypeStruct((2 * NP, 1), jnp.float32),
    ],
)


def _mid_body(agg_ref, ri_ref, sc_ref, w_ref, b_ref, out_ref):
    a = agg_ref[...].reshape(_RB, D).astype(jnp.float32) * ri_ref[...]
    h = jnp.dot(a, w_ref[...], preferred_element_type=jnp.float32)
    h = jnp.maximum(h + b_ref[...], 0.0) * sc_ref[...]
    out_ref[...] = h.astype(jnp.bfloat16).reshape(_RB * D)


_mid_call = pl.pallas_call(
    _mid_body,
    grid=(2 * NP // _RB,),
    in_specs=[
        pl.BlockSpec((_RB * D,), lambda i: (i,)),
        pl.BlockSpec((_RB, 1), lambda i: (i, 0)),
        pl.BlockSpec((_RB, 1), lambda i: (i, 0)),
        pl.BlockSpec((D, D), lambda i: (0, 0)),
        pl.BlockSpec((1, D), lambda i: (0, 0)),
    ],
    out_specs=pl.BlockSpec((_RB * D,), lambda i: (i,)),
    out_shape=jax.ShapeDtypeStruct((2 * NP * D,), jnp.bfloat16),
)


def _fin_body(h_ref, g1_ref, g2_ref, desc_ref,
              c1w_ref, c1b_ref, c2w_ref, c2b_ref, c3w_ref, c3b_ref,
              c4w_ref, c4b_ref, out_ref):
    h2d = h_ref[...].reshape(2 * NP, D)
    iota = lax.broadcasted_iota(jnp.int32, (1, B), 1)

    def pool(g_ref, rows):
        m = (g_ref[...] == iota).astype(jnp.float32)
        s = lax.dot_general(m, rows, (((0,), (0,)), ((), ())),
                            preferred_element_type=jnp.float32)
        cnt = jnp.sum(m, axis=0)[:, None]
        return s / jnp.maximum(cnt, 1.0)

    hg1 = pool(g1_ref, h2d[0:N].astype(jnp.float32))
    hg2 = pool(g2_ref, h2d[NP:NP + N].astype(jnp.float32))

    c1w = c1w_ref[...]
    z = (jnp.dot(hg1, c1w[0:D], preferred_element_type=jnp.float32)
         + jnp.dot(hg2, c1w[D:2 * D], preferred_element_type=jnp.float32)
         + jnp.dot(desc_ref[...], c1w[2 * D:], preferred_element_type=jnp.float32)
         + c1b_ref[...])
    z = jnp.maximum(z, 0.0)
    z = jnp.maximum(jnp.dot(z, c2w_ref[...],
                            preferred_element_type=jnp.float32) + c2b_ref[...], 0.0)
    z = jnp.maximum(jnp.dot(z, c3w_ref[...],
                            preferred_element_type=jnp.float32) + c3b_ref[...], 0.0)
    out_ref[...] = jnp.dot(z, c4w_ref[...],
                           preferred_element_type=jnp.float32) + c4b_ref[...]


_fin_call = pl.pallas_call(
    _fin_body,
    out_shape=jax.ShapeDtypeStruct((B, 1), jnp.float32),
)


def kernel(x1, x2, edge_index1, edge_index2, graph_ids1, graph_ids2,
           descriptors, W1, b1, W2, b2, C1W, C1b, C2W, C2b, C3W, C3b,
           C4W, C4b):
    e1r = edge_index1.reshape(2, _RAW_CHUNKS, CH)
    e2r = edge_index2.reshape(2, _RAW_CHUNKS, CH)
    deg, srcs, dsts = _deg_kernel(e1r, e2r)

    xpad = jnp.pad(jnp.stack([x1, x2]), ((0, 0), (0, NP - N), (0, 0)))
    dego4 = deg[:, :NP].reshape(2, NP // _RB, 1, _RB)
    degi4 = deg[:, NP:].reshape(2, NP // _RB, 1, _RB)
    xn, ri, ro = _prep_call(xpad, dego4, degi4)

    wl = jnp.stack([W1, W2])
    bl = jnp.stack([b1.reshape(1, D), b2.reshape(1, D)])
    sc = jnp.stack([ro, jnp.ones_like(ro)])

    def layer(h, per):
        w, b_, s_ = per
        agg = _agg_kernel(h.reshape(2 * NP, D), srcs, dsts)
        return _mid_call(agg.reshape(2 * NP * D), ri, s_, w, b_), None

    h, _ = lax.scan(layer, xn, (wl, bl, sc))

    return _fin_call(h, graph_ids1.reshape(N, 1), graph_ids2.reshape(N, 1),
                     descriptors, C1W, C1b.reshape(1, 2 * D + 16),
                     C2W, C2b.reshape(1, D), C3W, C3b.reshape(1, D),
                     C4W, C4b.reshape(1, 1))
